# baseline (speedup 1.0000x reference)
import jax
import jax.numpy as jnp
from jax import lax
from jax.experimental import pallas as pl
from jax.experimental.pallas import tpu as pltpu
from jax.experimental.pallas import tpu_sc as plsc

NUM_ROWS = 100000
EMBED_K = 64
BATCH = 16384

_INFO = plsc.get_sparse_core_info()
_NC, _NS, _L = _INFO.num_cores, _INFO.num_subcores, _INFO.num_lanes
_KPT = EMBED_K // (_NC * _NS)
_STRIP = 8192
_UNROLL = 8


def _mf_col_body(xt_hbm, wt_hbm, ht_hbm, part_hbm, col_v, wa_v, idx_v, sem):
    c = lax.axis_index("c")
    s = lax.axis_index("s")

    for kk in range(_KPT):
        k = c * (EMBED_K // _NC) + s * _KPT + kk

        pltpu.sync_copy(wt_hbm.at[k], col_v)
        for st in range(BATCH // _STRIP):
            pltpu.sync_copy(xt_hbm.at[0, pl.ds(st * _STRIP, _STRIP)], idx_v)

            def wbody(j, _):
                for q in range(_UNROLL):
                    off = j * _L * _UNROLL + q * _L
                    u = idx_v[pl.ds(off, _L)]
                    wa_v[pl.ds(st * _STRIP + off, _L)] = plsc.load_gather(
                        col_v, [u])
                return 0

            lax.fori_loop(0, _STRIP // (_L * _UNROLL), wbody, 0)

        pltpu.sync_copy(ht_hbm.at[k], col_v)
        for st in range(BATCH // _STRIP):
            pltpu.sync_copy(xt_hbm.at[1, pl.ds(st * _STRIP, _STRIP)], idx_v)

            def hbody(j, _):
                for q in range(_UNROLL):
                    off = j * _L * _UNROLL + q * _L
                    base = st * _STRIP + off
                    v = idx_v[pl.ds(off, _L)]
                    hv = plsc.load_gather(col_v, [v])
                    wa_v[pl.ds(base, _L)] = wa_v[pl.ds(base, _L)] * hv
                return 0

            lax.fori_loop(0, _STRIP // (_L * _UNROLL), hbody, 0)

        pltpu.sync_copy(wa_v, part_hbm.at[k])


def _combine_body(p_ref, o_ref):
    o_ref[...] = jnp.sum(p_ref[...], axis=0)


@jax.jit
def kernel(x, W, H):
    xt = x.astype(jnp.int32).T
    wt = W.T
    ht = H.T

    mf = pl.kernel(
        _mf_col_body,
        out_type=jax.ShapeDtypeStruct((EMBED_K, BATCH), jnp.float32),
        mesh=plsc.VectorSubcoreMesh(core_axis_name="c", subcore_axis_name="s"),
        scratch_types=[
            pltpu.VMEM((NUM_ROWS,), jnp.float32),
            pltpu.VMEM((BATCH,), jnp.float32),
            pltpu.VMEM((_STRIP,), jnp.int32),
            pltpu.SemaphoreType.DMA,
        ],
        compiler_params=pltpu.CompilerParams(
            needs_layout_passes=False, use_tc_tiling_on_sc=True),
    )
    part = mf(xt, wt, ht)

    out = pl.pallas_call(
        _combine_body,
        out_shape=jax.ShapeDtypeStruct((BATCH,), jnp.float32),
    )(part)
    return out


# speedup vs baseline: 1.2145x; 1.2145x over previous
import jax
import jax.numpy as jnp
from jax import lax
from jax.experimental import pallas as pl
from jax.experimental.pallas import tpu as pltpu
from jax.experimental.pallas import tpu_sc as plsc

NUM_ROWS = 100000
EMBED_K = 64
BATCH = 16384

_INFO = plsc.get_sparse_core_info()
_NC, _NS, _L = _INFO.num_cores, _INFO.num_subcores, _INFO.num_lanes
_KPT = EMBED_K // (_NC * _NS)
_STRIP = 8192
_UNROLL = 8


def _mf_col_body(xt_hbm, wt_hbm, ht_hbm, part_hbm, col_v, wa_v, idx_v, sem):
    c = lax.axis_index("c")
    s = lax.axis_index("s")

    for kk in range(_KPT):
        k = c * (EMBED_K // _NC) + s * _KPT + kk

        pltpu.sync_copy(wt_hbm.at[k], col_v)
        for st in range(BATCH // _STRIP):
            pltpu.sync_copy(xt_hbm.at[0, pl.ds(st * _STRIP, _STRIP)], idx_v)

            @plsc.parallel_loop(0, _STRIP // _L, unroll=_UNROLL)
            def wbody(j):
                off = j * _L
                u = idx_v[pl.ds(off, _L)]
                wa_v[pl.ds(st * _STRIP + off, _L)] = plsc.load_gather(
                    col_v, [u])

        pltpu.sync_copy(ht_hbm.at[k], col_v)
        for st in range(BATCH // _STRIP):
            pltpu.sync_copy(xt_hbm.at[1, pl.ds(st * _STRIP, _STRIP)], idx_v)

            @plsc.parallel_loop(0, _STRIP // _L, unroll=_UNROLL)
            def hbody(j):
                off = j * _L
                base = st * _STRIP + off
                v = idx_v[pl.ds(off, _L)]
                hv = plsc.load_gather(col_v, [v])
                wa_v[pl.ds(base, _L)] = wa_v[pl.ds(base, _L)] * hv

        pltpu.sync_copy(wa_v, part_hbm.at[k])


def _combine_body(p_ref, o_ref):
    o_ref[...] = jnp.sum(p_ref[...], axis=0)


@jax.jit
def kernel(x, W, H):
    xt = x.astype(jnp.int32).T
    wt = W.T
    ht = H.T

    mf = pl.kernel(
        _mf_col_body,
        out_type=jax.ShapeDtypeStruct((EMBED_K, BATCH), jnp.float32),
        mesh=plsc.VectorSubcoreMesh(core_axis_name="c", subcore_axis_name="s"),
        scratch_types=[
            pltpu.VMEM((NUM_ROWS,), jnp.float32),
            pltpu.VMEM((BATCH,), jnp.float32),
            pltpu.VMEM((_STRIP,), jnp.int32),
            pltpu.SemaphoreType.DMA,
        ],
        compiler_params=pltpu.CompilerParams(
            needs_layout_passes=False, use_tc_tiling_on_sc=True),
    )
    part = mf(xt, wt, ht)

    out = pl.pallas_call(
        _combine_body,
        out_shape=jax.ShapeDtypeStruct((BATCH,), jnp.float32),
    )(part)
    return out
